# baseline (speedup 1.0000x reference)
import jax
import jax.numpy as jnp
from jax import lax
from jax.experimental import pallas as pl

B = 16
L = 512
D = 256
NH = 4
DH = 64
T = 1024


def _body(t_ref, x_ref, u_ref, A_ref, W0_ref, Wu0_ref, b0_ref,
          W1_ref, Wu1_ref, b1_ref, out_ref):
    t_idx = t_ref[0, 0, :]
    x = x_ref[0]
    u = u_ref[0]
    A = A_ref[...]

    cols = lax.broadcasted_iota(jnp.int32, (L, T), 1)
    P = (cols == t_idx[:, None]).astype(jnp.bfloat16)
    rows = lax.broadcasted_iota(jnp.int32, (T, L), 0)
    Pt = (rows == t_idx[None, :]).astype(jnp.bfloat16)

    PA = jnp.dot(P, A.astype(jnp.bfloat16),
                 preferred_element_type=jnp.float32
                 ).astype(jnp.bfloat16)
    Asub = jnp.dot(PA, Pt,
                   preferred_element_type=jnp.float32)
    Asub_b = Asub.astype(jnp.bfloat16)

    r2 = lax.broadcasted_iota(jnp.int32, (L, L), 0)
    c2 = lax.broadcasted_iota(jnp.int32, (L, L), 1)
    diag = jnp.sum(jnp.where(r2 == c2, Asub, 0.0), axis=0)
    S = jnp.sum(diag)
    deg = jnp.sum(Asub, axis=1) + S + 1e-8
    inv_deg = (1.0 / deg)[:, None]

    ub = u.astype(jnp.bfloat16)

    def layer(xin, W_ref, Wu_ref, b_ref):
        cd = jnp.sum(diag[:, None] * xin, axis=0)
        g = jnp.dot(Asub_b, xin.astype(jnp.bfloat16),
                    preferred_element_type=jnp.float32)
        msg = (g + cd[None, :]) * inv_deg
        msg_b = msg.astype(jnp.bfloat16)
        hs = [jnp.dot(msg_b[:, h * DH:(h + 1) * DH],
                      W_ref[h].astype(jnp.bfloat16),
                      preferred_element_type=jnp.float32)
              for h in range(NH)]
        h = jnp.concatenate(hs, axis=1)
        U = jnp.dot(ub, Wu_ref[...].astype(jnp.bfloat16),
                    preferred_element_type=jnp.float32) + b_ref[0][None, :]
        return jnp.maximum(h + U, 0.0) + xin

    x1 = layer(x, W0_ref, Wu0_ref, b0_ref)
    x2 = layer(x1, W1_ref, Wu1_ref, b1_ref)
    out_ref[0, 0, :] = jnp.mean(x2, axis=0)


def kernel(user_id, event_type, enc_output, user_output, adjacent_matrix,
           W0, Wu0, b0, W1, Wu1, b1):
    t = event_type.astype(jnp.int32) - 1
    t = jnp.where(t < 0, t + 1000, t).reshape(B, 1, L)
    A_pad = jnp.pad(adjacent_matrix, ((0, T - 1000), (0, T - 1000)))
    b0r = b0.reshape(1, D)
    b1r = b1.reshape(1, D)

    grid = (B,)
    out = pl.pallas_call(
        _body,
        grid=grid,
        in_specs=[
            pl.BlockSpec((1, 1, L), lambda b: (b, 0, 0)),
            pl.BlockSpec((1, L, D), lambda b: (b, 0, 0)),
            pl.BlockSpec((1, L, D), lambda b: (b, 0, 0)),
            pl.BlockSpec((T, T), lambda b: (0, 0)),
            pl.BlockSpec((NH, DH, DH), lambda b: (0, 0, 0)),
            pl.BlockSpec((D, D), lambda b: (0, 0)),
            pl.BlockSpec((1, D), lambda b: (0, 0)),
            pl.BlockSpec((NH, DH, DH), lambda b: (0, 0, 0)),
            pl.BlockSpec((D, D), lambda b: (0, 0)),
            pl.BlockSpec((1, D), lambda b: (0, 0)),
        ],
        out_specs=pl.BlockSpec((1, 1, D), lambda b: (b, 0, 0)),
        out_shape=jax.ShapeDtypeStruct((B, 1, D), jnp.float32),
    )(t, enc_output, user_output, A_pad, W0, Wu0, b0r, W1, Wu1, b1r)
    return out.reshape(B, D)


# speedup vs baseline: 1.0830x; 1.0830x over previous
import jax
import jax.numpy as jnp
from jax import lax
from jax.experimental import pallas as pl

B = 16
L = 512
D = 256
NH = 4
DH = 64
T = 1024


def _body(t_ref, x_ref, u_ref, A_ref, Wb0_ref, Wu0_ref, b0_ref,
          Wb1_ref, Wu1_ref, b1_ref, out_ref):
    t_idx = t_ref[0, 0, :]
    x = x_ref[0]
    u = u_ref[0]
    A = A_ref[...]

    cols = lax.broadcasted_iota(jnp.int32, (L, T), 1)
    P = (cols == t_idx[:, None]).astype(jnp.bfloat16)
    rows = lax.broadcasted_iota(jnp.int32, (T, L), 0)
    Pt = (rows == t_idx[None, :]).astype(jnp.bfloat16)

    PA = jnp.dot(P, A, preferred_element_type=jnp.float32
                 ).astype(jnp.bfloat16)
    Asub = jnp.dot(PA, Pt,
                   preferred_element_type=jnp.float32)
    Asub_b = Asub.astype(jnp.bfloat16)

    r2 = lax.broadcasted_iota(jnp.int32, (L, L), 0)
    c2 = lax.broadcasted_iota(jnp.int32, (L, L), 1)
    diag = jnp.sum(jnp.where(r2 == c2, Asub, 0.0), axis=0)
    S = jnp.sum(diag)
    deg = jnp.sum(Asub, axis=1) + S + 1e-8
    inv_deg = (1.0 / deg)[:, None]

    ub = u.astype(jnp.bfloat16)

    def layer(xin, Wb_ref, Wu_ref, b_ref):
        cd = jnp.sum(diag[:, None] * xin, axis=0)
        g = jnp.dot(Asub_b, xin.astype(jnp.bfloat16),
                    preferred_element_type=jnp.float32)
        msg = (g + cd[None, :]) * inv_deg
        h = jnp.dot(msg.astype(jnp.bfloat16), Wb_ref[...],
                    preferred_element_type=jnp.float32)
        U = jnp.dot(ub, Wu_ref[...],
                    preferred_element_type=jnp.float32) + b_ref[0][None, :]
        return jnp.maximum(h + U, 0.0) + xin

    x1 = layer(x, Wb0_ref, Wu0_ref, b0_ref)
    x2 = layer(x1, Wb1_ref, Wu1_ref, b1_ref)
    out_ref[0, 0, :] = jnp.mean(x2, axis=0)


def kernel(user_id, event_type, enc_output, user_output, adjacent_matrix,
           W0, Wu0, b0, W1, Wu1, b1):
    t = event_type.astype(jnp.int32) - 1
    t = jnp.where(t < 0, t + 1000, t).reshape(B, 1, L)
    A_pad = jnp.pad(adjacent_matrix, ((0, T - 1000), (0, T - 1000))
                    ).astype(jnp.bfloat16)
    b0r = b0.reshape(1, D)
    b1r = b1.reshape(1, D)
    hmask = (jnp.arange(NH)[:, None, None, None] ==
             jnp.arange(NH)[None, None, :, None])
    Wb0 = jnp.where(hmask, W0[:, :, None, :], 0.0)
    Wb0 = Wb0.reshape(D, D).astype(jnp.bfloat16)
    Wb1 = jnp.where(hmask, W1[:, :, None, :], 0.0)
    Wb1 = Wb1.reshape(D, D).astype(jnp.bfloat16)
    Wu0b = Wu0.astype(jnp.bfloat16)
    Wu1b = Wu1.astype(jnp.bfloat16)

    grid = (B,)
    out = pl.pallas_call(
        _body,
        grid=grid,
        in_specs=[
            pl.BlockSpec((1, 1, L), lambda b: (b, 0, 0)),
            pl.BlockSpec((1, L, D), lambda b: (b, 0, 0)),
            pl.BlockSpec((1, L, D), lambda b: (b, 0, 0)),
            pl.BlockSpec((T, T), lambda b: (0, 0)),
            pl.BlockSpec((D, D), lambda b: (0, 0)),
            pl.BlockSpec((D, D), lambda b: (0, 0)),
            pl.BlockSpec((1, D), lambda b: (0, 0)),
            pl.BlockSpec((D, D), lambda b: (0, 0)),
            pl.BlockSpec((D, D), lambda b: (0, 0)),
            pl.BlockSpec((1, D), lambda b: (0, 0)),
        ],
        out_specs=pl.BlockSpec((1, 1, D), lambda b: (b, 0, 0)),
        out_shape=jax.ShapeDtypeStruct((B, 1, D), jnp.float32),
    )(t, enc_output, user_output, A_pad, Wb0, Wu0b, b0r, Wb1, Wu1b, b1r)
    return out.reshape(B, D)
